# baseline (speedup 1.0000x reference)
import jax
import jax.numpy as jnp
from jax.experimental import pallas as pl
from jax.experimental.pallas import tpu as pltpu

_N_CHUNKS = 8


def _overlap_copy(w_ref, o_ref, buf, *sems):
    rows = o_ref.shape[0]
    chunk = rows // _N_CHUNKS
    isems = sems[:_N_CHUNKS]
    osems = sems[_N_CHUNKS:]
    in_cps = []
    out_cps = []
    for j in range(_N_CHUNKS):
        sl = pl.ds(j * chunk, chunk)
        in_cps.append(pltpu.make_async_copy(w_ref.at[sl, :], buf.at[sl, :], isems[j]))
        out_cps.append(pltpu.make_async_copy(buf.at[sl, :], o_ref.at[sl, :], osems[j]))
    in_cps[0].start()
    in_cps[1].start()
    for j in range(_N_CHUNKS):
        in_cps[j].wait()
        out_cps[j].start()
        if j + 2 < _N_CHUNKS:
            in_cps[j + 2].start()
    for j in range(_N_CHUNKS):
        out_cps[j].wait()


def kernel(x, W):
    seq_len = x.shape[1]
    n_model = W.shape[1]
    out = pl.pallas_call(
        _overlap_copy,
        in_specs=[pl.BlockSpec(memory_space=pl.ANY)],
        out_specs=pl.BlockSpec(memory_space=pl.ANY),
        out_shape=jax.ShapeDtypeStruct((seq_len, n_model), W.dtype),
        scratch_shapes=[pltpu.VMEM((seq_len, n_model), W.dtype)]
        + [pltpu.SemaphoreType.DMA] * (2 * _N_CHUNKS),
    )(W)
    return out


# speedup vs baseline: 1.1742x; 1.1742x over previous
import jax
import jax.numpy as jnp
from jax.experimental import pallas as pl
from jax.experimental.pallas import tpu as pltpu

_N_CHUNKS = 4


def _overlap_copy(w_ref, o_ref, buf, *sems):
    rows = o_ref.shape[0]
    chunk = rows // _N_CHUNKS
    isems = sems[:_N_CHUNKS]
    osems = sems[_N_CHUNKS:]
    in_cps = []
    out_cps = []
    for j in range(_N_CHUNKS):
        sl = pl.ds(j * chunk, chunk)
        in_cps.append(pltpu.make_async_copy(w_ref.at[sl, :], buf.at[sl, :], isems[j]))
        out_cps.append(pltpu.make_async_copy(buf.at[sl, :], o_ref.at[sl, :], osems[j]))
    for j in range(_N_CHUNKS):
        in_cps[j].start()
    for j in range(_N_CHUNKS):
        in_cps[j].wait()
        out_cps[j].start()
    for j in range(_N_CHUNKS):
        out_cps[j].wait()


def kernel(x, W):
    seq_len = x.shape[1]
    n_model = W.shape[1]
    out = pl.pallas_call(
        _overlap_copy,
        in_specs=[pl.BlockSpec(memory_space=pl.ANY)],
        out_specs=pl.BlockSpec(memory_space=pl.ANY),
        out_shape=jax.ShapeDtypeStruct((seq_len, n_model), W.dtype),
        scratch_shapes=[pltpu.VMEM((seq_len, n_model), W.dtype)]
        + [pltpu.SemaphoreType.DMA] * (2 * _N_CHUNKS),
    )(W)
    return out


# speedup vs baseline: 1.1812x; 1.0060x over previous
import jax
import jax.numpy as jnp
from jax.experimental import pallas as pl
from jax.experimental.pallas import tpu as pltpu

_N_CHUNKS = 2


def _overlap_copy(w_ref, o_ref, buf, *sems):
    rows = o_ref.shape[0]
    chunk = rows // _N_CHUNKS
    isems = sems[:_N_CHUNKS]
    osems = sems[_N_CHUNKS:]
    in_cps = []
    out_cps = []
    for j in range(_N_CHUNKS):
        sl = pl.ds(j * chunk, chunk)
        in_cps.append(pltpu.make_async_copy(w_ref.at[sl, :], buf.at[sl, :], isems[j]))
        out_cps.append(pltpu.make_async_copy(buf.at[sl, :], o_ref.at[sl, :], osems[j]))
    for j in range(_N_CHUNKS):
        in_cps[j].start()
    for j in range(_N_CHUNKS):
        in_cps[j].wait()
        out_cps[j].start()
    for j in range(_N_CHUNKS):
        out_cps[j].wait()


def kernel(x, W):
    seq_len = x.shape[1]
    n_model = W.shape[1]
    out = pl.pallas_call(
        _overlap_copy,
        in_specs=[pl.BlockSpec(memory_space=pl.ANY)],
        out_specs=pl.BlockSpec(memory_space=pl.ANY),
        out_shape=jax.ShapeDtypeStruct((seq_len, n_model), W.dtype),
        scratch_shapes=[pltpu.VMEM((seq_len, n_model), W.dtype)]
        + [pltpu.SemaphoreType.DMA] * (2 * _N_CHUNKS),
    )(W)
    return out
